# baseline (speedup 1.0000x reference)
import functools

import jax
import jax.numpy as jnp
from jax import lax
from jax.experimental import pallas as pl
from jax.experimental.pallas import tpu as pltpu
from jax.experimental.pallas import tpu_sc as plsc

N = 100000
D = 128
OUT_D = 3 * D + 3
S = 80
NSTEPS = N // S
NW = 32
TAIL = OUT_D - 16
NUM_ATOMIC = 119
NUM_VALENCE = 8
NUM_HYBRID = 8

_mesh = plsc.VectorSubcoreMesh(core_axis_name="c", subcore_axis_name="s")


@functools.partial(
    pl.kernel,
    mesh=_mesh,
    compiler_params=pltpu.CompilerParams(needs_layout_passes=False),
    out_type=jax.ShapeDtypeStruct((N, OUT_D), jnp.float32),
    scratch_types=[
        pltpu.VMEM((NUM_ATOMIC, D), jnp.float32),
        pltpu.VMEM((NUM_VALENCE, D), jnp.float32),
        pltpu.VMEM((NUM_HYBRID, D), jnp.float32),
        pltpu.VMEM((3, S), jnp.int32),
        pltpu.VMEM((3, S), jnp.int32),
        pltpu.VMEM((S, 3 * D), jnp.float32),
        pltpu.VMEM((S, 3 * D), jnp.float32),
        pltpu.SemaphoreType.DMA,
        pltpu.SemaphoreType.DMA,
        pltpu.SemaphoreType.DMA,
        pltpu.SemaphoreType.DMA,
    ],
)
def _bands(atomic_h, valence_h, hyb_h, ta_h, tv_h, th_h,
           out_h, tab_a, tab_v, tab_h, idx0, idx1, block0, block1,
           sem_in0, sem_in1, sem_o0, sem_o1):
    wid = lax.axis_index("s") * 2 + lax.axis_index("c")
    nsteps_w = (NSTEPS - 1 - wid) // NW + 1
    ita = lax.iota(jnp.int32, 16)
    z16 = jnp.zeros((16,), jnp.int32)
    o16 = jnp.full((16,), 1, jnp.int32)
    t16 = jnp.full((16,), 2, jnp.int32)

    idxs = (idx0, idx1)
    blocks = (block0, block1)
    sem_in = (sem_in0, sem_in1)
    sem_o = (sem_o0, sem_o1)

    pltpu.sync_copy(ta_h, tab_a)
    pltpu.sync_copy(tv_h, tab_v)
    pltpu.sync_copy(th_h, tab_h)

    def issue_in(step, b):
        base = step * S
        pltpu.async_copy(atomic_h.at[pl.ds(base, S)], idxs[b].at[0], sem_in[b])
        pltpu.async_copy(valence_h.at[pl.ds(base, S)], idxs[b].at[1], sem_in[b])
        pltpu.async_copy(hyb_h.at[pl.ds(base, S)], idxs[b].at[2], sem_in[b])

    def wait_in(b):
        pltpu.make_async_copy(atomic_h.at[pl.ds(0, S)], idxs[b].at[0], sem_in[b]).wait()
        pltpu.make_async_copy(valence_h.at[pl.ds(0, S)], idxs[b].at[1], sem_in[b]).wait()
        pltpu.make_async_copy(hyb_h.at[pl.ds(0, S)], idxs[b].at[2], sem_in[b]).wait()

    def wait_out(b):
        pltpu.make_async_copy(
            blocks[b], out_h.at[pl.ds(0, S), pl.ds(0, 3 * D)], sem_o[b]).wait()

    def do_step(j, b):
        step = wid + NW * j
        base = step * S
        blk = blocks[b]
        wait_in(b)

        @pl.when(j + 1 < nsteps_w)
        def _():
            issue_in(wid + NW * (j + 1), 1 - b)

        @pl.when(j >= 2)
        def _():
            wait_out(b)

        def fgroup(g, c):
            for j16 in range(16):
                r = g * 16 + j16
                rfull = jnp.full((16,), r, jnp.int32)
                ra = plsc.load_gather(idxs[b], [z16, rfull])
                rv = plsc.load_gather(idxs[b], [o16, rfull])
                rh = plsc.load_gather(idxs[b], [t16, rfull])
                for tab, rowv, band in ((tab_a, ra, 0), (tab_v, rv, D),
                                        (tab_h, rh, 2 * D)):
                    for c16 in range(8):
                        vals = plsc.load_gather(tab, [rowv, c16 * 16 + ita])
                        blk[r, pl.ds(band + c16 * 16, 16)] = vals
            return c

        lax.fori_loop(0, S // 16, fgroup, 0)
        pltpu.async_copy(blk, out_h.at[pl.ds(base, S), pl.ds(0, 3 * D)], sem_o[b])

    @pl.when(nsteps_w > 0)
    def _():
        issue_in(wid, 0)

    def pair_body(k, c):
        for b in (0, 1):
            j = 2 * k + b

            @pl.when(j < nsteps_w)
            def _():
                do_step(j, b)

        return c

    lax.fori_loop(0, (nsteps_w + 1) // 2, pair_body, 0)
    for b in (0, 1):
        @pl.when(nsteps_w > b)
        def _():
            wait_out(b)


@functools.partial(
    pl.kernel,
    mesh=_mesh,
    out_type=jax.ShapeDtypeStruct((N, OUT_D), jnp.float32),
    scratch_types=[
        pltpu.VMEM((3 * S,), jnp.float32),
        pltpu.VMEM((3 * S,), jnp.float32),
        pltpu.VMEM((S, OUT_D), jnp.float32),
        pltpu.VMEM((S, OUT_D), jnp.float32),
        pltpu.SemaphoreType.DMA,
        pltpu.SemaphoreType.DMA,
        pltpu.SemaphoreType.DMA,
        pltpu.SemaphoreType.DMA,
        pltpu.SemaphoreType.DMA,
        pltpu.SemaphoreType.DMA,
    ],
)
def _feats(bands_h, fc_h, ar_h, re_h, out_h, pack0, pack1, block0, block1,
           sem_in0, sem_in1, sem_r0, sem_r1, sem_o0, sem_o1):
    wid = lax.axis_index("s") * 2 + lax.axis_index("c")
    nsteps_w = (NSTEPS - 1 - wid) // NW + 1
    ita = lax.iota(jnp.int32, 16)
    is13 = ita == 13
    is14 = ita == 14
    is15 = ita == 15

    packs = (pack0, pack1)
    blocks = (block0, block1)
    sem_in = (sem_in0, sem_in1)
    sem_r = (sem_r0, sem_r1)
    sem_o = (sem_o0, sem_o1)

    def issue_in(step, b):
        base = step * S
        pltpu.async_copy(bands_h.at[pl.ds(base, S)], blocks[b], sem_r[b])
        pltpu.async_copy(fc_h.at[pl.ds(base, S)], packs[b].at[pl.ds(0, S)], sem_in[b])
        pltpu.async_copy(ar_h.at[pl.ds(base, S)], packs[b].at[pl.ds(S, S)], sem_in[b])
        pltpu.async_copy(re_h.at[pl.ds(base, S)], packs[b].at[pl.ds(2 * S, S)], sem_in[b])

    def wait_in(b):
        pltpu.make_async_copy(bands_h.at[pl.ds(0, S)], blocks[b], sem_r[b]).wait()
        pltpu.make_async_copy(fc_h.at[pl.ds(0, S)], packs[b].at[pl.ds(0, S)], sem_in[b]).wait()
        pltpu.make_async_copy(ar_h.at[pl.ds(0, S)], packs[b].at[pl.ds(S, S)], sem_in[b]).wait()
        pltpu.make_async_copy(re_h.at[pl.ds(0, S)], packs[b].at[pl.ds(2 * S, S)], sem_in[b]).wait()

    def wait_out(b):
        pltpu.make_async_copy(blocks[b], out_h.at[pl.ds(0, S)], sem_o[b]).wait()

    def do_step(j, b):
        step = wid + NW * j
        base = step * S
        blk = blocks[b]
        pack = packs[b]

        wait_in(b)

        @pl.when(j + 1 < nsteps_w)
        def _():
            @pl.when(j >= 1)
            def _():
                wait_out(1 - b)

            issue_in(wid + NW * (j + 1), 1 - b)

        def fgroup(g, c):
            fcv = pack[pl.ds(g * 16, 16)]
            arv = pack[pl.ds(S + g * 16, 16)]
            rev = pack[pl.ds(2 * S + g * 16, 16)]
            for j16 in range(16):
                r = g * 16 + j16
                cur = blk[r, pl.ds(TAIL, 16)]
                rep = blk[r, pl.ds(360, 16)]
                fcb = jnp.full((16,), fcv[j16], jnp.float32)
                arb = jnp.full((16,), arv[j16], jnp.float32)
                reb = jnp.full((16,), rev[j16], jnp.float32)
                tail = jnp.where(
                    is13, fcb, jnp.where(is14, arb, jnp.where(is15, reb, cur)))
                blk[r, pl.ds(TAIL, 16)] = tail
                blk[r, pl.ds(360, 16)] = rep
            return c

        lax.fori_loop(0, S // 16, fgroup, 0)
        pltpu.async_copy(blk, out_h.at[pl.ds(base, S)], sem_o[b])

    @pl.when(nsteps_w > 0)
    def _():
        issue_in(wid, 0)

    def pair_body(k, c):
        for b in (0, 1):
            j = 2 * k + b

            @pl.when(j < nsteps_w)
            def _():
                do_step(j, b)

        return c

    lax.fori_loop(0, (nsteps_w + 1) // 2, pair_body, 0)
    for b in (0, 1):
        @pl.when(nsteps_w > b)
        def _():
            wait_out(b)


def kernel(atomic, valence, formal_charge, aromatic, hybridization,
           radical_electrons, atomic_table, valence_table, hybridization_table):
    bands = _bands(atomic, valence, hybridization,
                   atomic_table, valence_table, hybridization_table)
    return _feats(bands, formal_charge, aromatic, radical_electrons)


# speedup vs baseline: 1.3731x; 1.3731x over previous
import functools

import jax
import jax.numpy as jnp
from jax import lax
from jax.experimental import pallas as pl
from jax.experimental.pallas import tpu as pltpu
from jax.experimental.pallas import tpu_sc as plsc

N = 100000
D = 128
OUT_D = 3 * D + 3
S = 80
NSTEPS = N // S
NW = 32
TAIL = OUT_D - 16
NUM_ATOMIC = 119
NUM_VALENCE = 8
NUM_HYBRID = 8

_mesh = plsc.VectorSubcoreMesh(core_axis_name="c", subcore_axis_name="s")


@functools.partial(
    pl.kernel,
    mesh=_mesh,
    compiler_params=pltpu.CompilerParams(needs_layout_passes=False),
    out_type=jax.ShapeDtypeStruct((N, OUT_D), jnp.float32),
    scratch_types=[
        pltpu.VMEM((NUM_ATOMIC, D), jnp.float32),
        pltpu.VMEM((NUM_VALENCE, D), jnp.float32),
        pltpu.VMEM((NUM_HYBRID, D), jnp.float32),
        pltpu.VMEM((3, S), jnp.int32),
        pltpu.VMEM((3, S), jnp.int32),
        pltpu.VMEM((S, 3 * D), jnp.float32),
        pltpu.VMEM((S, 3 * D), jnp.float32),
        pltpu.SemaphoreType.DMA,
        pltpu.SemaphoreType.DMA,
        pltpu.SemaphoreType.DMA,
        pltpu.SemaphoreType.DMA,
    ],
)
def _bands(atomic_h, valence_h, hyb_h, ta_h, tv_h, th_h,
           out_h, tab_a, tab_v, tab_h, idx0, idx1, block0, block1,
           sem_in0, sem_in1, sem_o0, sem_o1):
    wid = lax.axis_index("s") * 2 + lax.axis_index("c")
    nsteps_w = (NSTEPS - 1 - wid) // NW + 1
    ita = lax.iota(jnp.int32, 16)
    z16 = jnp.zeros((16,), jnp.int32)
    o16 = jnp.full((16,), 1, jnp.int32)
    t16 = jnp.full((16,), 2, jnp.int32)

    idxs = (idx0, idx1)
    blocks = (block0, block1)
    sem_in = (sem_in0, sem_in1)
    sem_o = (sem_o0, sem_o1)

    pltpu.sync_copy(ta_h, tab_a)
    pltpu.sync_copy(tv_h, tab_v)
    pltpu.sync_copy(th_h, tab_h)

    def issue_in(step, b):
        base = step * S
        pltpu.async_copy(atomic_h.at[pl.ds(base, S)], idxs[b].at[0], sem_in[b])
        pltpu.async_copy(valence_h.at[pl.ds(base, S)], idxs[b].at[1], sem_in[b])
        pltpu.async_copy(hyb_h.at[pl.ds(base, S)], idxs[b].at[2], sem_in[b])

    def wait_in(b):
        pltpu.make_async_copy(atomic_h.at[pl.ds(0, S)], idxs[b].at[0], sem_in[b]).wait()
        pltpu.make_async_copy(valence_h.at[pl.ds(0, S)], idxs[b].at[1], sem_in[b]).wait()
        pltpu.make_async_copy(hyb_h.at[pl.ds(0, S)], idxs[b].at[2], sem_in[b]).wait()

    def wait_out(b):
        pltpu.make_async_copy(
            blocks[b], out_h.at[pl.ds(0, S), pl.ds(0, 3 * D)], sem_o[b]).wait()

    def do_step(j, b):
        step = wid + NW * j
        base = step * S
        blk = blocks[b]
        wait_in(b)

        @pl.when(j + 1 < nsteps_w)
        def _():
            issue_in(wid + NW * (j + 1), 1 - b)

        @pl.when(j >= 2)
        def _():
            wait_out(b)

        def fgroup(g, c):
            for j16 in range(16):
                r = g * 16 + j16
                rfull = jnp.full((16,), r, jnp.int32)
                ra = plsc.load_gather(idxs[b], [z16, rfull])
                rv = plsc.load_gather(idxs[b], [o16, rfull])
                rh = plsc.load_gather(idxs[b], [t16, rfull])
                gathered = []
                for tab, rowv, band in ((tab_a, ra, 0), (tab_v, rv, D),
                                        (tab_h, rh, 2 * D)):
                    for c16 in range(8):
                        vals = plsc.load_gather(tab, [rowv, c16 * 16 + ita])
                        gathered.append((band + c16 * 16, vals))
                for col, vals in gathered:
                    blk[r, pl.ds(col, 16)] = vals
            return c

        lax.fori_loop(0, S // 16, fgroup, 0)
        pltpu.async_copy(blk, out_h.at[pl.ds(base, S), pl.ds(0, 3 * D)], sem_o[b])

    @pl.when(nsteps_w > 0)
    def _():
        issue_in(wid, 0)

    def pair_body(k, c):
        for b in (0, 1):
            j = 2 * k + b

            @pl.when(j < nsteps_w)
            def _():
                do_step(j, b)

        return c

    lax.fori_loop(0, (nsteps_w + 1) // 2, pair_body, 0)
    for b in (0, 1):
        @pl.when(nsteps_w > b)
        def _():
            wait_out(b)


@functools.partial(
    pl.kernel,
    mesh=_mesh,
    out_type=jax.ShapeDtypeStruct((N, OUT_D), jnp.float32),
    scratch_types=[
        pltpu.VMEM((3 * S,), jnp.float32),
        pltpu.VMEM((3 * S,), jnp.float32),
        pltpu.VMEM((S, OUT_D), jnp.float32),
        pltpu.VMEM((S, OUT_D), jnp.float32),
        pltpu.SemaphoreType.DMA,
        pltpu.SemaphoreType.DMA,
        pltpu.SemaphoreType.DMA,
        pltpu.SemaphoreType.DMA,
        pltpu.SemaphoreType.DMA,
        pltpu.SemaphoreType.DMA,
    ],
)
def _feats(bands_h, fc_h, ar_h, re_h, out_h, pack0, pack1, block0, block1,
           sem_in0, sem_in1, sem_r0, sem_r1, sem_o0, sem_o1):
    wid = lax.axis_index("s") * 2 + lax.axis_index("c")
    nsteps_w = (NSTEPS - 1 - wid) // NW + 1
    ita = lax.iota(jnp.int32, 16)
    is13 = ita == 13
    is14 = ita == 14
    is15 = ita == 15

    packs = (pack0, pack1)
    blocks = (block0, block1)
    sem_in = (sem_in0, sem_in1)
    sem_r = (sem_r0, sem_r1)
    sem_o = (sem_o0, sem_o1)

    def issue_in(step, b):
        base = step * S
        pltpu.async_copy(bands_h.at[pl.ds(base, S)], blocks[b], sem_r[b])
        pltpu.async_copy(fc_h.at[pl.ds(base, S)], packs[b].at[pl.ds(0, S)], sem_in[b])
        pltpu.async_copy(ar_h.at[pl.ds(base, S)], packs[b].at[pl.ds(S, S)], sem_in[b])
        pltpu.async_copy(re_h.at[pl.ds(base, S)], packs[b].at[pl.ds(2 * S, S)], sem_in[b])

    def wait_in(b):
        pltpu.make_async_copy(bands_h.at[pl.ds(0, S)], blocks[b], sem_r[b]).wait()
        pltpu.make_async_copy(fc_h.at[pl.ds(0, S)], packs[b].at[pl.ds(0, S)], sem_in[b]).wait()
        pltpu.make_async_copy(ar_h.at[pl.ds(0, S)], packs[b].at[pl.ds(S, S)], sem_in[b]).wait()
        pltpu.make_async_copy(re_h.at[pl.ds(0, S)], packs[b].at[pl.ds(2 * S, S)], sem_in[b]).wait()

    def wait_out(b):
        pltpu.make_async_copy(blocks[b], out_h.at[pl.ds(0, S)], sem_o[b]).wait()

    def do_step(j, b):
        step = wid + NW * j
        base = step * S
        blk = blocks[b]
        pack = packs[b]

        wait_in(b)

        @pl.when(j + 1 < nsteps_w)
        def _():
            @pl.when(j >= 1)
            def _():
                wait_out(1 - b)

            issue_in(wid + NW * (j + 1), 1 - b)

        def fgroup(g, c):
            fcv = pack[pl.ds(g * 16, 16)]
            arv = pack[pl.ds(S + g * 16, 16)]
            rev = pack[pl.ds(2 * S + g * 16, 16)]
            for j16 in range(16):
                r = g * 16 + j16
                cur = blk[r, pl.ds(TAIL, 16)]
                rep = blk[r, pl.ds(360, 16)]
                fcb = jnp.full((16,), fcv[j16], jnp.float32)
                arb = jnp.full((16,), arv[j16], jnp.float32)
                reb = jnp.full((16,), rev[j16], jnp.float32)
                tail = jnp.where(
                    is13, fcb, jnp.where(is14, arb, jnp.where(is15, reb, cur)))
                blk[r, pl.ds(TAIL, 16)] = tail
                blk[r, pl.ds(360, 16)] = rep
            return c

        lax.fori_loop(0, S // 16, fgroup, 0)
        pltpu.async_copy(blk, out_h.at[pl.ds(base, S)], sem_o[b])

    @pl.when(nsteps_w > 0)
    def _():
        issue_in(wid, 0)

    def pair_body(k, c):
        for b in (0, 1):
            j = 2 * k + b

            @pl.when(j < nsteps_w)
            def _():
                do_step(j, b)

        return c

    lax.fori_loop(0, (nsteps_w + 1) // 2, pair_body, 0)
    for b in (0, 1):
        @pl.when(nsteps_w > b)
        def _():
            wait_out(b)


def kernel(atomic, valence, formal_charge, aromatic, hybridization,
           radical_electrons, atomic_table, valence_table, hybridization_table):
    bands = _bands(atomic, valence, hybridization,
                   atomic_table, valence_table, hybridization_table)
    return _feats(bands, formal_charge, aromatic, radical_electrons)
